# baseline (speedup 1.0000x reference)
import functools

import jax
import jax.numpy as jnp
from jax import lax
from jax.experimental import pallas as pl
from jax.experimental.pallas import tpu as pltpu
from jax.experimental.pallas import tpu_sc as plsc

VOCAB = 1000000
EMBED_DIM = 32
CLASS_NUM = 9
BATCH = 16384
SEQ_LEN = 200

PAD_DIM = 16
NW = 32
ROWS_PER_W = BATCH // NW
IDX_PER_W = ROWS_PER_W * SEQ_LEN
CHUNK_IDX = 3200
CHUNK_ROWS = CHUNK_IDX // SEQ_LEN
N_GATHER = CHUNK_IDX // 128
N_CHUNK = IDX_PER_W // CHUNK_IDX


_TC_ROWS = 1000


def _tc_body(a_ref, w_ref, b_ref, o_ref):
    o_ref[...] = (
        jnp.dot(a_ref[...], w_ref[...], preferred_element_type=jnp.float32)
        + b_ref[...]
    )


def _transform_table(table, W, b):
    scale = jnp.float32(1.0 / SEQ_LEN)
    Wp = jnp.zeros((EMBED_DIM, PAD_DIM), jnp.float32).at[:, :CLASS_NUM].set(W)
    bp = jnp.zeros((PAD_DIM,), jnp.float32).at[:CLASS_NUM].set(b)
    Wbig = jnp.kron(jnp.eye(4, dtype=jnp.float32), Wp * scale)
    bbig = jnp.tile(bp * scale, 4)[None, :]
    t4 = table.reshape(VOCAB // 4, 4 * EMBED_DIM)
    out4 = pl.pallas_call(
        _tc_body,
        grid=(VOCAB // 4 // (2 * _TC_ROWS),),
        in_specs=[
            pl.BlockSpec((2 * _TC_ROWS, 128), lambda i: (i, 0)),
            pl.BlockSpec((128, 64), lambda i: (0, 0)),
            pl.BlockSpec((1, 64), lambda i: (0, 0)),
        ],
        out_specs=pl.BlockSpec((2 * _TC_ROWS, 64), lambda i: (i, 0)),
        out_shape=jax.ShapeDtypeStruct((VOCAB // 4, 64), jnp.float32),
    )(t4, Wbig, bbig)
    return out4.reshape(VOCAB, PAD_DIM)


def _sc_body(t2_hbm, x_hbm, out_hbm, idx_v, gbuf, out_v, sem):
    wid = lax.axis_index("s") * 2 + lax.axis_index("c")
    idx_base = wid * IDX_PER_W
    out_row_base = wid * (ROWS_PER_W * PAD_DIM // 128)

    @pl.loop(0, N_CHUNK)
    def _chunk(s):
        pltpu.sync_copy(
            x_hbm.at[pl.ds(idx_base + s * CHUNK_IDX, CHUNK_IDX)], idx_v
        )
        copies = [
            pltpu.async_copy(
                t2_hbm.at[idx_v.at[pl.ds(j * 128, 128)]],
                gbuf.at[pl.ds(j * 128, 128)],
                sem,
            )
            for j in range(N_GATHER)
        ]
        for c in copies:
            c.wait()
        for r in range(CHUNK_ROWS):
            base = r * SEQ_LEN

            def acc_body(i, accs, base=base):
                a0, a1, a2, a3 = accs
                k = base + i * 4
                a0 = a0 + gbuf[k, :]
                a1 = a1 + gbuf[k + 1, :]
                a2 = a2 + gbuf[k + 2, :]
                a3 = a3 + gbuf[k + 3, :]
                return (a0, a1, a2, a3)

            z = jnp.zeros((PAD_DIM,), jnp.float32)
            a0, a1, a2, a3 = lax.fori_loop(0, SEQ_LEN // 4, acc_body,
                                           (z, z, z, z))
            out_v[s * 2 + r // 8, pl.ds((r % 8) * PAD_DIM, PAD_DIM)] = (
                (a0 + a1) + (a2 + a3)
            )

    pltpu.sync_copy(out_v, out_hbm.at[pl.ds(out_row_base, ROWS_PER_W * PAD_DIM // 128)])


@functools.partial(
    pl.kernel,
    out_type=jax.ShapeDtypeStruct((BATCH * PAD_DIM // 128, 128), jnp.float32),
    mesh=plsc.VectorSubcoreMesh(core_axis_name="c", subcore_axis_name="s"),
    scratch_types=[
        pltpu.VMEM((CHUNK_IDX,), jnp.int32),
        pltpu.VMEM((CHUNK_IDX, PAD_DIM), jnp.float32),
        pltpu.VMEM((ROWS_PER_W * PAD_DIM // 128, 128), jnp.float32),
        pltpu.SemaphoreType.DMA,
    ],
    compiler_params=pltpu.CompilerParams(use_tc_tiling_on_sc=False),
)
def _sc_gather_sum(t2p_hbm, x_hbm, out_hbm, idx_v, gbuf, out_v, sem):
    _sc_body(t2p_hbm, x_hbm, out_hbm, idx_v, gbuf, out_v, sem)


def kernel(x, table, W, b):
    t2 = _transform_table(table, W, b)
    x1 = x.astype(jnp.int32).reshape(BATCH * SEQ_LEN)
    out_packed = _sc_gather_sum(t2, x1)
    return out_packed.reshape(BATCH, PAD_DIM)[:, :CLASS_NUM]


# speedup vs baseline: 1.1881x; 1.1881x over previous
import functools

import jax
import jax.numpy as jnp
from jax import lax
from jax.experimental import pallas as pl
from jax.experimental.pallas import tpu as pltpu
from jax.experimental.pallas import tpu_sc as plsc

VOCAB = 1000000
EMBED_DIM = 32
CLASS_NUM = 9
BATCH = 16384
SEQ_LEN = 200

PAD_DIM = 16
NW = 32
ROWS_PER_W = BATCH // NW
IDX_PER_W = ROWS_PER_W * SEQ_LEN
CHUNK_IDX = 3200
CHUNK_ROWS = CHUNK_IDX // SEQ_LEN
N_GATHER = CHUNK_IDX // 128
N_CHUNK = IDX_PER_W // CHUNK_IDX


_TC_ROWS = 8192


def _tc_body(a_ref, w_ref, b_ref, o_ref):
    o_ref[...] = (
        jax.lax.dot_general(
            a_ref[...], w_ref[...],
            (((0,), (0,)), ((), ())),
            preferred_element_type=jnp.float32,
        )
        + b_ref[...]
    )


def _transform_table(table, W, b):
    scale = jnp.float32(1.0 / SEQ_LEN)
    Wp = jnp.zeros((EMBED_DIM, PAD_DIM), jnp.float32).at[:, :CLASS_NUM].set(W)
    bp = jnp.zeros((PAD_DIM,), jnp.float32).at[:CLASS_NUM].set(b)
    tT = table.T
    return pl.pallas_call(
        _tc_body,
        grid=(pl.cdiv(VOCAB, _TC_ROWS),),
        in_specs=[
            pl.BlockSpec((EMBED_DIM, _TC_ROWS), lambda i: (0, i)),
            pl.BlockSpec((EMBED_DIM, PAD_DIM), lambda i: (0, 0)),
            pl.BlockSpec((1, PAD_DIM), lambda i: (0, 0)),
        ],
        out_specs=pl.BlockSpec((_TC_ROWS, PAD_DIM), lambda i: (i, 0)),
        out_shape=jax.ShapeDtypeStruct((VOCAB, PAD_DIM), jnp.float32),
    )(tT, Wp * scale, (bp * scale)[None, :])


def _sc_body(t2_hbm, x_hbm, out_hbm, idx_v, gbuf, out_v, sem):
    wid = lax.axis_index("s") * 2 + lax.axis_index("c")
    idx_base = wid * IDX_PER_W
    out_row_base = wid * (ROWS_PER_W * PAD_DIM // 128)

    @pl.loop(0, N_CHUNK)
    def _chunk(s):
        pltpu.sync_copy(
            x_hbm.at[pl.ds(idx_base + s * CHUNK_IDX, CHUNK_IDX)], idx_v
        )
        copies = [
            pltpu.async_copy(
                t2_hbm.at[idx_v.at[pl.ds(j * 128, 128)]],
                gbuf.at[pl.ds(j * 128, 128)],
                sem,
            )
            for j in range(N_GATHER)
        ]
        for c in copies:
            c.wait()
        for r in range(CHUNK_ROWS):
            base = r * SEQ_LEN

            def acc_body(i, accs, base=base):
                a0, a1, a2, a3 = accs
                k = base + i * 4
                a0 = a0 + gbuf[k, :]
                a1 = a1 + gbuf[k + 1, :]
                a2 = a2 + gbuf[k + 2, :]
                a3 = a3 + gbuf[k + 3, :]
                return (a0, a1, a2, a3)

            z = jnp.zeros((PAD_DIM,), jnp.float32)
            a0, a1, a2, a3 = lax.fori_loop(0, SEQ_LEN // 4, acc_body,
                                           (z, z, z, z))
            out_v[s * 2 + r // 8, pl.ds((r % 8) * PAD_DIM, PAD_DIM)] = (
                (a0 + a1) + (a2 + a3)
            )

    pltpu.sync_copy(out_v, out_hbm.at[pl.ds(out_row_base, ROWS_PER_W * PAD_DIM // 128)])


@functools.partial(
    pl.kernel,
    out_type=jax.ShapeDtypeStruct((BATCH * PAD_DIM // 128, 128), jnp.float32),
    mesh=plsc.VectorSubcoreMesh(core_axis_name="c", subcore_axis_name="s"),
    scratch_types=[
        pltpu.VMEM((CHUNK_IDX,), jnp.int32),
        pltpu.VMEM((CHUNK_IDX, PAD_DIM), jnp.float32),
        pltpu.VMEM((ROWS_PER_W * PAD_DIM // 128, 128), jnp.float32),
        pltpu.SemaphoreType.DMA,
    ],
    compiler_params=pltpu.CompilerParams(use_tc_tiling_on_sc=False),
)
def _sc_gather_sum(t2p_hbm, x_hbm, out_hbm, idx_v, gbuf, out_v, sem):
    _sc_body(t2p_hbm, x_hbm, out_hbm, idx_v, gbuf, out_v, sem)


def kernel(x, table, W, b):
    t2 = _transform_table(table, W, b)
    x1 = x.astype(jnp.int32).reshape(BATCH * SEQ_LEN)
    out_packed = _sc_gather_sum(t2, x1)
    return out_packed.reshape(BATCH, PAD_DIM)[:, :CLASS_NUM]


# speedup vs baseline: 1.3752x; 1.1575x over previous
import functools

import jax
import jax.numpy as jnp
from jax import lax
from jax.experimental import pallas as pl
from jax.experimental.pallas import tpu as pltpu
from jax.experimental.pallas import tpu_sc as plsc

VOCAB = 1000000
EMBED_DIM = 32
CLASS_NUM = 9
BATCH = 16384
SEQ_LEN = 200

PAD_DIM = 16
NW = 32
ROWS_PER_W = BATCH // NW
IDX_PER_W = ROWS_PER_W * SEQ_LEN
CHUNK_IDX = 3200
CHUNK_ROWS = CHUNK_IDX // SEQ_LEN
N_GATHER = CHUNK_IDX // 128
N_CHUNK = IDX_PER_W // CHUNK_IDX


_TC_ROWS = 8192


def _tc_body(a_ref, w_ref, b_ref, o_ref):
    o_ref[...] = (
        jax.lax.dot_general(
            a_ref[...], w_ref[...],
            (((0,), (0,)), ((), ())),
            preferred_element_type=jnp.float32,
        )
        + b_ref[...]
    )


def _transform_table(table, W, b):
    scale = jnp.float32(1.0 / SEQ_LEN)
    Wp = jnp.zeros((EMBED_DIM, PAD_DIM), jnp.float32).at[:, :CLASS_NUM].set(W)
    bp = jnp.zeros((PAD_DIM,), jnp.float32).at[:CLASS_NUM].set(b)
    tT = table.T
    return pl.pallas_call(
        _tc_body,
        grid=(pl.cdiv(VOCAB, _TC_ROWS),),
        in_specs=[
            pl.BlockSpec((EMBED_DIM, _TC_ROWS), lambda i: (0, i)),
            pl.BlockSpec((EMBED_DIM, PAD_DIM), lambda i: (0, 0)),
            pl.BlockSpec((1, PAD_DIM), lambda i: (0, 0)),
        ],
        out_specs=pl.BlockSpec((_TC_ROWS, PAD_DIM), lambda i: (i, 0)),
        out_shape=jax.ShapeDtypeStruct((VOCAB, PAD_DIM), jnp.float32),
    )(tT, Wp * scale, (bp * scale)[None, :])


def _sc_body(t2_hbm, x_hbm, out_hbm, idx_v0, idx_v1, gbuf0, gbuf1, out_v,
             sem0, sem1):
    wid = lax.axis_index("s") * 2 + lax.axis_index("c")
    idx_base = wid * IDX_PER_W
    out_row_base = wid * (ROWS_PER_W * PAD_DIM // 128)

    def gather_copies(s, idx_v, gbuf, sem):
        return [
            pltpu.make_async_copy(
                t2_hbm.at[idx_v.at[pl.ds(j * 128, 128)]],
                gbuf.at[pl.ds(j * 128, 128)],
                sem,
            )
            for j in range(N_GATHER)
        ]

    def fire(s, idx_v, gbuf, sem):
        pltpu.sync_copy(
            x_hbm.at[pl.ds(idx_base + s * CHUNK_IDX, CHUNK_IDX)], idx_v
        )
        for c in gather_copies(s, idx_v, gbuf, sem):
            c.start()

    def drain(s, idx_v, gbuf, sem):
        for c in gather_copies(s, idx_v, gbuf, sem):
            c.wait()

    def accumulate(s, gbuf):
        for r in range(CHUNK_ROWS):
            base = r * SEQ_LEN

            def acc_body(i, accs, base=base):
                a0, a1, a2, a3 = accs
                k = base + i * 8
                a0 = a0 + gbuf[k, :]
                a1 = a1 + gbuf[k + 1, :]
                a2 = a2 + gbuf[k + 2, :]
                a3 = a3 + gbuf[k + 3, :]
                a0 = a0 + gbuf[k + 4, :]
                a1 = a1 + gbuf[k + 5, :]
                a2 = a2 + gbuf[k + 6, :]
                a3 = a3 + gbuf[k + 7, :]
                return (a0, a1, a2, a3)

            z = jnp.zeros((PAD_DIM,), jnp.float32)
            a0, a1, a2, a3 = lax.fori_loop(0, SEQ_LEN // 8, acc_body,
                                           (z, z, z, z))
            out_v[s * 2 + r // 8, pl.ds((r % 8) * PAD_DIM, PAD_DIM)] = (
                (a0 + a1) + (a2 + a3)
            )

    fire(0, idx_v0, gbuf0, sem0)

    @pl.loop(0, N_CHUNK - 2, step=2)
    def _chunk(g):
        fire(g + 1, idx_v1, gbuf1, sem1)
        drain(g, idx_v0, gbuf0, sem0)
        accumulate(g, gbuf0)
        fire(g + 2, idx_v0, gbuf0, sem0)
        drain(g + 1, idx_v1, gbuf1, sem1)
        accumulate(g + 1, gbuf1)

    fire(N_CHUNK - 1, idx_v1, gbuf1, sem1)
    drain(N_CHUNK - 2, idx_v0, gbuf0, sem0)
    accumulate(N_CHUNK - 2, gbuf0)
    drain(N_CHUNK - 1, idx_v1, gbuf1, sem1)
    accumulate(N_CHUNK - 1, gbuf1)

    pltpu.sync_copy(out_v, out_hbm.at[pl.ds(out_row_base, ROWS_PER_W * PAD_DIM // 128)])


@functools.partial(
    pl.kernel,
    out_type=jax.ShapeDtypeStruct((BATCH * PAD_DIM // 128, 128), jnp.float32),
    mesh=plsc.VectorSubcoreMesh(core_axis_name="c", subcore_axis_name="s"),
    scratch_types=[
        pltpu.VMEM((CHUNK_IDX,), jnp.int32),
        pltpu.VMEM((CHUNK_IDX,), jnp.int32),
        pltpu.VMEM((CHUNK_IDX, PAD_DIM), jnp.float32),
        pltpu.VMEM((CHUNK_IDX, PAD_DIM), jnp.float32),
        pltpu.VMEM((ROWS_PER_W * PAD_DIM // 128, 128), jnp.float32),
        pltpu.SemaphoreType.DMA,
        pltpu.SemaphoreType.DMA,
    ],
    compiler_params=pltpu.CompilerParams(use_tc_tiling_on_sc=False),
)
def _sc_gather_sum(t2_hbm, x_hbm, out_hbm, idx_v0, idx_v1, gbuf0, gbuf1,
                   out_v, sem0, sem1):
    _sc_body(t2_hbm, x_hbm, out_hbm, idx_v0, idx_v1, gbuf0, gbuf1, out_v,
             sem0, sem1)


def kernel(x, table, W, b):
    t2 = _transform_table(table, W, b)
    x1 = x.astype(jnp.int32).reshape(BATCH * SEQ_LEN)
    out_packed = _sc_gather_sum(t2, x1)
    return out_packed.reshape(BATCH, PAD_DIM)[:, :CLASS_NUM]


# speedup vs baseline: 1.3781x; 1.0021x over previous
import functools

import jax
import jax.numpy as jnp
from jax import lax
from jax.experimental import pallas as pl
from jax.experimental.pallas import tpu as pltpu
from jax.experimental.pallas import tpu_sc as plsc

VOCAB = 1000000
EMBED_DIM = 32
CLASS_NUM = 9
BATCH = 16384
SEQ_LEN = 200

PAD_DIM = 16
NW = 32
ROWS_PER_W = BATCH // NW
IDX_PER_W = ROWS_PER_W * SEQ_LEN
CHUNK_IDX = 3200
CHUNK_ROWS = CHUNK_IDX // SEQ_LEN
N_GATHER = CHUNK_IDX // 128
N_CHUNK = IDX_PER_W // CHUNK_IDX


_TC_ROWS = 8192


def _tc_body(a_ref, w_ref, b_ref, o_ref):
    o_ref[...] = (
        jax.lax.dot_general(
            a_ref[...], w_ref[...],
            (((0,), (0,)), ((), ())),
            preferred_element_type=jnp.float32,
            precision=jax.lax.Precision.DEFAULT,
        )
        + b_ref[...]
    )


def _transform_table(table, W, b):
    scale = jnp.float32(1.0 / SEQ_LEN)
    Wp = jnp.zeros((EMBED_DIM, PAD_DIM), jnp.float32).at[:, :CLASS_NUM].set(W)
    bp = jnp.zeros((PAD_DIM,), jnp.float32).at[:CLASS_NUM].set(b)
    tT = table.T
    return pl.pallas_call(
        _tc_body,
        grid=(pl.cdiv(VOCAB, _TC_ROWS),),
        in_specs=[
            pl.BlockSpec((EMBED_DIM, _TC_ROWS), lambda i: (0, i)),
            pl.BlockSpec((EMBED_DIM, PAD_DIM), lambda i: (0, 0)),
            pl.BlockSpec((1, PAD_DIM), lambda i: (0, 0)),
        ],
        out_specs=pl.BlockSpec((_TC_ROWS, PAD_DIM), lambda i: (i, 0)),
        out_shape=jax.ShapeDtypeStruct((VOCAB, PAD_DIM), jnp.float32),
    )(tT, Wp * scale, (bp * scale)[None, :])


def _sc_body(t2_hbm, x_hbm, out_hbm, idx_v0, idx_v1, gbuf0, gbuf1, out_v,
             sem0, sem1):
    wid = lax.axis_index("s") * 2 + lax.axis_index("c")
    idx_base = wid * IDX_PER_W
    out_row_base = wid * (ROWS_PER_W * PAD_DIM // 128)

    def gather_copies(s, idx_v, gbuf, sem):
        return [
            pltpu.make_async_copy(
                t2_hbm.at[idx_v.at[pl.ds(j * 128, 128)]],
                gbuf.at[pl.ds(j * 128, 128)],
                sem,
            )
            for j in range(N_GATHER)
        ]

    def fire(s, idx_v, gbuf, sem):
        pltpu.sync_copy(
            x_hbm.at[pl.ds(idx_base + s * CHUNK_IDX, CHUNK_IDX)], idx_v
        )
        for c in gather_copies(s, idx_v, gbuf, sem):
            c.start()

    def drain(s, idx_v, gbuf, sem):
        for c in gather_copies(s, idx_v, gbuf, sem):
            c.wait()

    def accumulate(s, gbuf):
        for r in range(CHUNK_ROWS):
            base = r * SEQ_LEN

            def acc_body(i, accs, base=base):
                a0, a1, a2, a3 = accs
                k = base + i * 8
                a0 = a0 + gbuf[k, :]
                a1 = a1 + gbuf[k + 1, :]
                a2 = a2 + gbuf[k + 2, :]
                a3 = a3 + gbuf[k + 3, :]
                a0 = a0 + gbuf[k + 4, :]
                a1 = a1 + gbuf[k + 5, :]
                a2 = a2 + gbuf[k + 6, :]
                a3 = a3 + gbuf[k + 7, :]
                return (a0, a1, a2, a3)

            z = jnp.zeros((PAD_DIM,), jnp.float32)
            a0, a1, a2, a3 = lax.fori_loop(0, SEQ_LEN // 8, acc_body,
                                           (z, z, z, z))
            out_v[s * 2 + r // 8, pl.ds((r % 8) * PAD_DIM, PAD_DIM)] = (
                (a0 + a1) + (a2 + a3)
            )

    fire(0, idx_v0, gbuf0, sem0)

    @pl.loop(0, N_CHUNK - 2, step=2)
    def _chunk(g):
        fire(g + 1, idx_v1, gbuf1, sem1)
        drain(g, idx_v0, gbuf0, sem0)
        accumulate(g, gbuf0)
        fire(g + 2, idx_v0, gbuf0, sem0)
        drain(g + 1, idx_v1, gbuf1, sem1)
        accumulate(g + 1, gbuf1)

    fire(N_CHUNK - 1, idx_v1, gbuf1, sem1)
    drain(N_CHUNK - 2, idx_v0, gbuf0, sem0)
    accumulate(N_CHUNK - 2, gbuf0)
    drain(N_CHUNK - 1, idx_v1, gbuf1, sem1)
    accumulate(N_CHUNK - 1, gbuf1)

    pltpu.sync_copy(out_v, out_hbm.at[pl.ds(out_row_base, ROWS_PER_W * PAD_DIM // 128)])


@functools.partial(
    pl.kernel,
    out_type=jax.ShapeDtypeStruct((BATCH * PAD_DIM // 128, 128), jnp.float32),
    mesh=plsc.VectorSubcoreMesh(core_axis_name="c", subcore_axis_name="s"),
    scratch_types=[
        pltpu.VMEM((CHUNK_IDX,), jnp.int32),
        pltpu.VMEM((CHUNK_IDX,), jnp.int32),
        pltpu.VMEM((CHUNK_IDX, PAD_DIM), jnp.float32),
        pltpu.VMEM((CHUNK_IDX, PAD_DIM), jnp.float32),
        pltpu.VMEM((ROWS_PER_W * PAD_DIM // 128, 128), jnp.float32),
        pltpu.SemaphoreType.DMA,
        pltpu.SemaphoreType.DMA,
    ],
    compiler_params=pltpu.CompilerParams(use_tc_tiling_on_sc=False),
)
def _sc_gather_sum(t2_hbm, x_hbm, out_hbm, idx_v0, idx_v1, gbuf0, gbuf1,
                   out_v, sem0, sem1):
    _sc_body(t2_hbm, x_hbm, out_hbm, idx_v0, idx_v1, gbuf0, gbuf1, out_v,
             sem0, sem1)


def kernel(x, table, W, b):
    t2 = _transform_table(table, W, b)
    x1 = x.astype(jnp.int32).reshape(BATCH * SEQ_LEN)
    out_packed = _sc_gather_sum(t2, x1)
    return out_packed.reshape(BATCH, PAD_DIM)[:, :CLASS_NUM]


# speedup vs baseline: 2.7211x; 1.9746x over previous
import functools

import jax
import jax.numpy as jnp
from jax import lax
from jax.experimental import pallas as pl
from jax.experimental.pallas import tpu as pltpu
from jax.experimental.pallas import tpu_sc as plsc

VOCAB = 1000000
EMBED_DIM = 32
CLASS_NUM = 9
BATCH = 16384
SEQ_LEN = 200

PAD_DIM = 16
NW = 32
ROWS_PER_W = BATCH // NW
IDX_PER_W = ROWS_PER_W * SEQ_LEN
CHUNK_IDX = 3200
CHUNK_ROWS = CHUNK_IDX // SEQ_LEN
N_GATHER = CHUNK_IDX // 128
N_CHUNK = IDX_PER_W // CHUNK_IDX


_TC_ROWS = 8192


def _tc_body(a_ref, w_ref, b_ref, o_ref):
    a = a_ref[...]
    r = o_ref.shape[0]
    acc = b_ref[...]
    for g in range(8):
        acc = acc + jax.lax.dot_general(
            a[:, g * r:(g + 1) * r],
            w_ref[g * EMBED_DIM:(g + 1) * EMBED_DIM, :],
            (((0,), (0,)), ((), ())),
            preferred_element_type=jnp.float32,
        )
    o_ref[...] = acc


def _transform_table(table, W, b):
    scale = jnp.float32(1.0 / SEQ_LEN)
    Wp = jnp.zeros((EMBED_DIM, PAD_DIM), jnp.float32).at[:, :CLASS_NUM].set(W)
    bp = jnp.zeros((PAD_DIM,), jnp.float32).at[:CLASS_NUM].set(b)
    tT = table.T
    Wbig = jnp.kron(jnp.eye(8, dtype=jnp.float32), Wp * scale)
    bbig = jnp.tile(bp * scale, 8)[None, :]
    n_blk = pl.cdiv(VOCAB, _TC_ROWS)
    out = pl.pallas_call(
        _tc_body,
        grid=(n_blk,),
        in_specs=[
            pl.BlockSpec((EMBED_DIM, _TC_ROWS), lambda i: (0, i)),
            pl.BlockSpec((8 * EMBED_DIM, 128), lambda i: (0, 0)),
            pl.BlockSpec((1, 128), lambda i: (0, 0)),
        ],
        out_specs=pl.BlockSpec((_TC_ROWS // 8, 128), lambda i: (i, 0)),
        out_shape=jax.ShapeDtypeStruct((n_blk * _TC_ROWS // 8, 128),
                                       jnp.float32),
    )(tT, Wbig, bbig)
    return out.reshape(n_blk * _TC_ROWS, PAD_DIM)


def _sc_body(t2_hbm, x_hbm, out_hbm, idx_v0, idx_v1, gbuf0, gbuf1, out_v,
             sem0, sem1):
    wid = lax.axis_index("s") * 2 + lax.axis_index("c")
    idx_base = wid * IDX_PER_W
    out_row_base = wid * (ROWS_PER_W * PAD_DIM // 128)

    def gather_copies(s, idx_v, gbuf, sem):
        return [
            pltpu.make_async_copy(
                t2_hbm.at[idx_v.at[pl.ds(j * 128, 128)]],
                gbuf.at[pl.ds(j * 128, 128)],
                sem,
            )
            for j in range(N_GATHER)
        ]

    def fire(s, idx_v, gbuf, sem):
        pltpu.sync_copy(
            x_hbm.at[pl.ds(idx_base + s * CHUNK_IDX, CHUNK_IDX)], idx_v
        )

        @pl.loop(0, CHUNK_IDX // 16)
        def _remap(k):
            v = idx_v[pl.ds(k * 16, 16)]
            idx_v[pl.ds(k * 16, 16)] = (
                ((v >> 13) << 13) + ((v & 1023) << 3) + ((v >> 10) & 7)
            )

        for c in gather_copies(s, idx_v, gbuf, sem):
            c.start()

    def drain(s, idx_v, gbuf, sem):
        for c in gather_copies(s, idx_v, gbuf, sem):
            c.wait()

    def accumulate(s, gbuf):
        for r in range(CHUNK_ROWS):
            base = r * SEQ_LEN

            def acc_body(i, accs, base=base):
                a0, a1, a2, a3 = accs
                k = base + i * 8
                a0 = a0 + gbuf[k, :]
                a1 = a1 + gbuf[k + 1, :]
                a2 = a2 + gbuf[k + 2, :]
                a3 = a3 + gbuf[k + 3, :]
                a0 = a0 + gbuf[k + 4, :]
                a1 = a1 + gbuf[k + 5, :]
                a2 = a2 + gbuf[k + 6, :]
                a3 = a3 + gbuf[k + 7, :]
                return (a0, a1, a2, a3)

            z = jnp.zeros((PAD_DIM,), jnp.float32)
            a0, a1, a2, a3 = lax.fori_loop(0, SEQ_LEN // 8, acc_body,
                                           (z, z, z, z))
            out_v[s * 2 + r // 8, pl.ds((r % 8) * PAD_DIM, PAD_DIM)] = (
                (a0 + a1) + (a2 + a3)
            )

    fire(0, idx_v0, gbuf0, sem0)

    @pl.loop(0, N_CHUNK - 2, step=2)
    def _chunk(g):
        fire(g + 1, idx_v1, gbuf1, sem1)
        drain(g, idx_v0, gbuf0, sem0)
        accumulate(g, gbuf0)
        fire(g + 2, idx_v0, gbuf0, sem0)
        drain(g + 1, idx_v1, gbuf1, sem1)
        accumulate(g + 1, gbuf1)

    fire(N_CHUNK - 1, idx_v1, gbuf1, sem1)
    drain(N_CHUNK - 2, idx_v0, gbuf0, sem0)
    accumulate(N_CHUNK - 2, gbuf0)
    drain(N_CHUNK - 1, idx_v1, gbuf1, sem1)
    accumulate(N_CHUNK - 1, gbuf1)

    pltpu.sync_copy(out_v, out_hbm.at[pl.ds(out_row_base, ROWS_PER_W * PAD_DIM // 128)])


@functools.partial(
    pl.kernel,
    out_type=jax.ShapeDtypeStruct((BATCH * PAD_DIM // 128, 128), jnp.float32),
    mesh=plsc.VectorSubcoreMesh(core_axis_name="c", subcore_axis_name="s"),
    scratch_types=[
        pltpu.VMEM((CHUNK_IDX,), jnp.int32),
        pltpu.VMEM((CHUNK_IDX,), jnp.int32),
        pltpu.VMEM((CHUNK_IDX, PAD_DIM), jnp.float32),
        pltpu.VMEM((CHUNK_IDX, PAD_DIM), jnp.float32),
        pltpu.VMEM((ROWS_PER_W * PAD_DIM // 128, 128), jnp.float32),
        pltpu.SemaphoreType.DMA,
        pltpu.SemaphoreType.DMA,
    ],
    compiler_params=pltpu.CompilerParams(use_tc_tiling_on_sc=False),
)
def _sc_gather_sum(t2_hbm, x_hbm, out_hbm, idx_v0, idx_v1, gbuf0, gbuf1,
                   out_v, sem0, sem1):
    _sc_body(t2_hbm, x_hbm, out_hbm, idx_v0, idx_v1, gbuf0, gbuf1, out_v,
             sem0, sem1)


def kernel(x, table, W, b):
    t2 = _transform_table(table, W, b)
    x1 = x.astype(jnp.int32).reshape(BATCH * SEQ_LEN)
    out_packed = _sc_gather_sum(t2, x1)
    return out_packed.reshape(BATCH, PAD_DIM)[:, :CLASS_NUM]
